# baseline (speedup 1.0000x reference)
import functools

import jax
import jax.numpy as jnp
import numpy as np
from jax.experimental import pallas as pl
from jax.experimental.pallas import tpu as pltpu


def _mm_body(x_ref, w_ref, b_ref, o_ref, *, act):
    acc = jnp.dot(x_ref[...], w_ref[...], preferred_element_type=jnp.float32)
    acc = acc + b_ref[...]
    if act == "gelu":
        acc = acc * 0.5 * (1.0 + jax.lax.erf(acc * np.float32(0.7071067811865476)))
    o_ref[...] = acc


def _mm(x, w, b, act=None, block_m=1000):
    m, k = x.shape
    n = w.shape[1]
    grid = (m // block_m,)
    return pl.pallas_call(
        functools.partial(_mm_body, act=act),
        grid=grid,
        in_specs=[
            pl.BlockSpec((block_m, k), lambda i: (i, 0)),
            pl.BlockSpec((k, n), lambda i: (0, 0)),
            pl.BlockSpec((1, n), lambda i: (0, 0)),
        ],
        out_specs=pl.BlockSpec((block_m, n), lambda i: (i, 0)),
        out_shape=jax.ShapeDtypeStruct((m, n), jnp.float32),
    )(x, w, b.reshape(1, n))


def _hgt_attention_jnp(feat_src, feat_dst, edges, kW, kb, qW, qb, vW, vb,
                       w_att, w_msg, mu, num_heads, n_dst):
    d_out = kW.shape[1]
    dk = d_out // num_heads
    k = (feat_src @ kW + kb).reshape(-1, num_heads, dk)
    v = (feat_src @ vW + vb).reshape(-1, num_heads, dk)
    q = (feat_dst @ qW + qb).reshape(-1, num_heads, dk)
    k = jnp.einsum('nhi,hij->nhj', k, w_att)
    v = jnp.einsum('nhi,hij->nhj', v, w_msg)
    src = edges[0]
    dst = edges[1]
    t = jnp.sum(q[dst] * k[src], axis=-1)
    attn = t * mu / np.sqrt(dk)
    m = jax.ops.segment_max(attn, dst, num_segments=n_dst)
    e = jnp.exp(attn - m[dst])
    s = jax.ops.segment_sum(e, dst, num_segments=n_dst)
    a = e / s[dst]
    h = jax.ops.segment_sum(v[src] * a[:, :, None], dst, num_segments=n_dst)
    return h.reshape(n_dst, d_out)


def _layer_norm(x, g, b, eps=1e-5):
    mu_ = jnp.mean(x, axis=-1, keepdims=True)
    var = jnp.mean((x - mu_) ** 2, axis=-1, keepdims=True)
    return (x - mu_) / jnp.sqrt(var + eps) * g + b


def kernel(feat_paper, feat_author, edges_writes, edges_writtenby, adapt_W,
           adapt_b, kW, kb, qW, qb, vW, vb, w_att, w_msg, mu, aW, ab, skip,
           norm_g, norm_b, predW, predb):
    H = mu.shape[-1]
    feats = [feat_paper, feat_author]
    hs = [_mm(feats[t], adapt_W[t], adapt_b[t], act="gelu") for t in range(2)]
    edge_lists = [edges_writes, edges_writtenby]
    etype_src = [1, 0]
    etype_dst = [0, 1]
    n_layers = kW.shape[0]
    for l in range(n_layers):
        agg = [None, None]
        for r in range(2):
            s_t = etype_src[r]
            d_t = etype_dst[r]
            agg[d_t] = _hgt_attention_jnp(
                hs[s_t], hs[d_t], edge_lists[r], kW[l, s_t], kb[l, s_t],
                qW[l, d_t], qb[l, d_t], vW[l, s_t], vb[l, s_t],
                w_att[l, r], w_msg[l, r], mu[l, r], H, hs[d_t].shape[0])
        new_hs = []
        for t in range(2):
            alpha = jax.nn.sigmoid(skip[l, t])
            trans = _mm(agg[t], aW[l, t], ab[l, t])
            out = alpha * trans + (1.0 - alpha) * hs[t]
            new_hs.append(_layer_norm(out, norm_g[l, t], norm_b[l, t]))
        hs = new_hs
    return hs[0] @ predW + predb
